# baseline (speedup 1.0000x reference)
import functools

import jax
import jax.numpy as jnp
from jax import lax
from jax.experimental import pallas as pl
from jax.experimental.pallas import tpu as pltpu
from jax.experimental.pallas import tpu_sc as plsc

VOCAB = 129280
HIDDEN = 2048
BATCH = 4
SEQ = 2048
NTOK = BATCH * SEQ

_NC = 2
_NS = 16
_NW = _NC * _NS

_B_PER_W = NTOK // _NW
_CH = 16
_NCHUNK = _B_PER_W // _CH
_NBUF = 3

_mesh = plsc.VectorSubcoreMesh(core_axis_name="c", subcore_axis_name="s")


@functools.partial(
    pl.kernel,
    mesh=_mesh,
    out_type=jax.ShapeDtypeStruct((NTOK, HIDDEN), jnp.float32),
    scratch_types=[
        pltpu.VMEM((_B_PER_W,), jnp.int32),
        pltpu.VMEM((_NBUF, _CH, HIDDEN), jnp.float32),
        pltpu.SemaphoreType.DMA,
        pltpu.SemaphoreType.DMA,
        pltpu.SemaphoreType.DMA,
        pltpu.SemaphoreType.DMA,
        pltpu.SemaphoreType.DMA,
        pltpu.SemaphoreType.DMA,
    ],
)
def _embed_lookup(ids_hbm, table_hbm, out_hbm, idx_v, rows_v, g0, g1, g2, o0, o1, o2):
    gsems = (g0, g1, g2)
    osems = (o0, o1, o2)
    wid = lax.axis_index("s") * _NC + lax.axis_index("c")
    base = wid * _B_PER_W
    pltpu.sync_copy(ids_hbm.at[pl.ds(base, _B_PER_W)], idx_v)

    def g_start(c, b):
        return pltpu.async_copy(
            table_hbm.at[idx_v.at[pl.ds(c * _CH, _CH)]], rows_v.at[b], gsems[b]
        )

    def o_start(c, b):
        return pltpu.async_copy(
            rows_v.at[b], out_hbm.at[pl.ds(base + c * _CH, _CH)], osems[b]
        )

    g = [None] * _NBUF
    o = [None] * _NBUF
    for n in range(2):
        g[n] = g_start(n, n)
    for c in range(_NCHUNK):
        b = c % _NBUF
        g[b].wait()
        o[b] = o_start(c, b)
        n = c + 2
        if n < _NCHUNK:
            bn = n % _NBUF
            if n >= _NBUF:
                o[bn].wait()
            g[bn] = g_start(n, bn)
    for b in range(_NBUF):
        o[b].wait()


def kernel(input_ids, embed_tokens):
    ids_flat = input_ids.reshape(-1).astype(jnp.int32)
    out = _embed_lookup(ids_flat, embed_tokens)
    return out.reshape(BATCH, SEQ, HIDDEN)


# speedup vs baseline: 1.0246x; 1.0246x over previous
import functools

import jax
import jax.numpy as jnp
from jax import lax
from jax.experimental import pallas as pl
from jax.experimental.pallas import tpu as pltpu
from jax.experimental.pallas import tpu_sc as plsc

VOCAB = 129280
HIDDEN = 2048
BATCH = 4
SEQ = 2048

_NC = 2
_NS = 16
_NW = _NC * _NS

_B_PER_W = BATCH * SEQ // _NW
_W_PER_ROW = SEQ // _B_PER_W
_CH = 16
_NCHUNK = _B_PER_W // _CH
_NBUF = 3

_mesh = plsc.VectorSubcoreMesh(core_axis_name="c", subcore_axis_name="s")


@functools.partial(
    pl.kernel,
    mesh=_mesh,
    out_type=jax.ShapeDtypeStruct((BATCH, SEQ, HIDDEN), jnp.float32),
    scratch_types=[
        pltpu.VMEM((_B_PER_W,), jnp.int32),
        pltpu.VMEM((_NBUF, _CH, HIDDEN), jnp.float32),
        pltpu.SemaphoreType.DMA,
        pltpu.SemaphoreType.DMA,
        pltpu.SemaphoreType.DMA,
        pltpu.SemaphoreType.DMA,
        pltpu.SemaphoreType.DMA,
        pltpu.SemaphoreType.DMA,
    ],
)
def _embed_lookup(ids_hbm, table_hbm, out_hbm, idx_v, rows_v, g0, g1, g2, o0, o1, o2):
    gsems = (g0, g1, g2)
    osems = (o0, o1, o2)
    wid = lax.axis_index("s") * _NC + lax.axis_index("c")
    bi = wid // _W_PER_ROW
    col = (wid % _W_PER_ROW) * _B_PER_W
    pltpu.sync_copy(ids_hbm.at[bi, pl.ds(col, _B_PER_W)], idx_v)

    def g_start(c, b):
        return pltpu.async_copy(
            table_hbm.at[idx_v.at[pl.ds(c * _CH, _CH)]], rows_v.at[b], gsems[b]
        )

    def o_start(c, b):
        return pltpu.async_copy(
            rows_v.at[b], out_hbm.at[bi, pl.ds(col + c * _CH, _CH)], osems[b]
        )

    g = [None] * _NBUF
    o = [None] * _NBUF
    for n in range(2):
        g[n] = g_start(n, n)
    for c in range(_NCHUNK):
        b = c % _NBUF
        n = c + 2
        if n < _NCHUNK:
            bn = n % _NBUF
            if n >= _NBUF:
                o[bn].wait()
            g[bn] = g_start(n, bn)
        g[b].wait()
        o[b] = o_start(c, b)
    for b in range(_NBUF):
        o[b].wait()


def kernel(input_ids, embed_tokens):
    return _embed_lookup(input_ids, embed_tokens)
